# baseline (speedup 1.0000x reference)
import numpy as np
import jax
import jax.numpy as jnp
from jax.experimental import pallas as pl

_ROWS = 32
_COLS = 1000000
_SUB = 8
_LANES = _COLS // _SUB

_K1 = np.uint32(42)
_KS = (np.uint32(0), np.uint32(42), np.uint32(42 ^ 0x1BD11BDA))
_ROT = ((13, 15, 26, 6), (17, 29, 16, 24))


def _row_kernel(x_ref, o_ref):
    i = pl.program_id(0)
    x = x_ref[0]
    sub = jax.lax.broadcasted_iota(jnp.uint32, (_SUB, _LANES), 0)
    lane = jax.lax.broadcasted_iota(jnp.uint32, (_SUB, _LANES), 1)
    n = jnp.uint32(i) * jnp.uint32(_COLS) + sub * jnp.uint32(_LANES) + lane
    x0 = jnp.zeros((_SUB, _LANES), jnp.uint32)
    x1 = n + _K1
    for it in range(5):
        for r in _ROT[it % 2]:
            x0 = x0 + x1
            x1 = (x1 << jnp.uint32(r)) | (x1 >> jnp.uint32(32 - r))
            x1 = x0 ^ x1
        x0 = x0 + _KS[(it + 1) % 3]
        x1 = x1 + _KS[(it + 2) % 3] + jnp.uint32(it + 1)
    bits = x0 ^ x1
    fb = jax.lax.bitcast_convert_type(
        (bits >> jnp.uint32(9)) | jnp.uint32(0x3F800000), jnp.float32)
    u = jnp.maximum(jnp.float32(1e-10),
                    (fb - jnp.float32(1.0)) + jnp.float32(1e-10))
    g = -jnp.log(-jnp.log(u))
    s = x + g
    m = jnp.max(s)
    e = jnp.exp(s - m)
    o_ref[0] = e * (jnp.float32(1.0) / jnp.sum(e))


def kernel(logits):
    x = logits.reshape(_ROWS, _SUB, _LANES)
    out = pl.pallas_call(
        _row_kernel,
        grid=(_ROWS,),
        in_specs=[pl.BlockSpec((1, _SUB, _LANES), lambda i: (i, 0, 0))],
        out_specs=pl.BlockSpec((1, _SUB, _LANES), lambda i: (i, 0, 0)),
        out_shape=jax.ShapeDtypeStruct((_ROWS, _SUB, _LANES), jnp.float32),
    )(x)
    return out.reshape(_ROWS, _COLS)
